# baseline (speedup 1.0000x reference)
import jax
import jax.numpy as jnp
from jax import lax
from jax.experimental import pallas as pl
from jax.experimental.pallas import tpu as pltpu
from jax.experimental.pallas import tpu_sc as plsc

B, S, H, V, P, T = 4, 512, 768, 30522, 512, 2
N = B * S
NW = 32
TPW = N // NW
PPW = S // NW
LANES = 16
NCH = H // LANES
UNROLL = 4


def _rsqrt(x):
    xb = lax.bitcast_convert_type(x, jnp.int32)
    yb = jnp.int32(0x5F3759DF) - lax.shift_right_logical(xb, 1)
    y = lax.bitcast_convert_type(yb, jnp.float32)
    for _ in range(3):
        y = y * (1.5 - 0.5 * x * y * y)
    return y


def _sc_body(ids_hbm, tt_hbm, word_hbm, pos_hbm, tte_hbm, gamma_hbm, beta_hbm,
             out_hbm, idx_v, tt_v, rows_v, pos01_v, tte_v, gamma_v, beta_v,
             sem, osem):
    c = lax.axis_index("c")
    s = lax.axis_index("s")
    wid = s * 2 + c
    pbase = wid * PPW

    ics = [pltpu.async_copy(ids_hbm.at[pl.ds(b * S + pbase, PPW)],
                            idx_v.at[pl.ds(b * PPW, PPW)], osem)
           for b in range(B)]
    gathers = []
    for b in range(B):
        ics[b].wait()
        gathers.append(
            pltpu.async_copy(word_hbm.at[idx_v.at[pl.ds(b * PPW, PPW)]],
                             rows_v.at[pl.ds(b * PPW, PPW)], sem))
    for b in range(B):
        pltpu.sync_copy(tt_hbm.at[pl.ds(b * S + pbase, PPW)],
                        tt_v.at[pl.ds(b * PPW, PPW)])
    pltpu.sync_copy(pos_hbm.at[pl.ds(pbase, PPW)], pos01_v.at[0])
    pltpu.sync_copy(pos_hbm.at[pl.ds(pbase, PPW)], pos01_v.at[1])
    pltpu.sync_copy(tte_hbm, tte_v)
    pltpu.sync_copy(gamma_hbm, gamma_v)
    pltpu.sync_copy(beta_hbm, beta_v)

    def precomb(k, _):
        @plsc.parallel_loop(0, NCH, unroll=8)
        def _(j):
            sl = pl.ds(j * LANES, LANES)
            pos01_v[0, k, sl] = pos01_v[0, k, sl] + tte_v[0, sl]
            pos01_v[1, k, sl] = pos01_v[1, k, sl] + tte_v[1, sl]
        return 0
    lax.fori_loop(0, PPW, precomb, 0)

    lane = jnp.arange(LANES, dtype=jnp.int32)
    zero = jnp.zeros((LANES,), jnp.float32)

    outs = []
    for b in range(B):
        gathers[b].wait()
        tt16 = tt_v[pl.ds(b * PPW, LANES)]

        def token_body(kk, _, b=b, tt16=tt16):
            ks = [4 * kk + u for u in range(4)]
            iis = [b * PPW + kq for kq in ks]
            tss = [jnp.sum(jnp.where(lane == kq, tt16, 0)) for kq in ks]

            def p1(j, car):
                accs = list(car)
                sl = pl.ds(j * LANES, LANES)
                for u in range(4):
                    e = rows_v[iis[u], sl] + pos01_v[tss[u], ks[u], sl]
                    rows_v[iis[u], sl] = e
                    accs[u] = accs[u] + e
                    accs[4 + u] = accs[4 + u] + e * e
                return tuple(accs)
            accs = plsc.parallel_loop(0, NCH, unroll=4,
                                      carry=(zero,) * 8)(p1)
            means = [jnp.sum(accs[u]) * (1.0 / H) for u in range(4)]
            rstds = [_rsqrt(jnp.sum(accs[4 + u]) * (1.0 / H)
                            - means[u] * means[u] + 1e-12) for u in range(4)]
            nmeans = [means[u] * rstds[u] for u in range(4)]

            @plsc.parallel_loop(0, NCH, unroll=4)
            def _(j):
                sl = pl.ds(j * LANES, LANES)
                g = gamma_v[sl]
                bt = beta_v[sl]
                for u in range(4):
                    rows_v[iis[u], sl] = (rows_v[iis[u], sl] * rstds[u]
                                          - nmeans[u]) * g + bt
            return 0
        lax.fori_loop(0, PPW // 4, token_body, 0)
        outs.append(pltpu.async_copy(rows_v.at[pl.ds(b * PPW, PPW)],
                                     out_hbm.at[pl.ds(b * S + pbase, PPW)],
                                     osem))
    for cp in outs:
        cp.wait()


@jax.jit
def kernel(input_ids, token_type_ids, word_embeddings, position_embeddings,
           token_type_embeddings, ln_gamma, ln_beta):
    mesh = plsc.VectorSubcoreMesh(core_axis_name="c", subcore_axis_name="s")
    k = pl.kernel(
        _sc_body,
        out_type=jax.ShapeDtypeStruct((N, H), jnp.float32),
        mesh=mesh,
        compiler_params=pltpu.CompilerParams(needs_layout_passes=False),
        scratch_types=[
            pltpu.VMEM((TPW,), jnp.int32),
            pltpu.VMEM((TPW + LANES,), jnp.int32),
            pltpu.VMEM((TPW, H), jnp.float32),
            pltpu.VMEM((T, PPW, H), jnp.float32),
            pltpu.VMEM((T, H), jnp.float32),
            pltpu.VMEM((H,), jnp.float32),
            pltpu.VMEM((H,), jnp.float32),
            pltpu.SemaphoreType.DMA,
            pltpu.SemaphoreType.DMA,
        ],
    )
    out = k(input_ids.reshape(N), token_type_ids.reshape(N),
            word_embeddings, position_embeddings, token_type_embeddings,
            ln_gamma, ln_beta)
    return out.reshape(B, S, H)
